# baseline (speedup 1.0000x reference)
import jax
import jax.numpy as jnp
from jax import lax
from jax.experimental import pallas as pl
from jax.experimental.pallas import tpu as pltpu
from jax.experimental.pallas import tpu_sc as plsc

N_NODES = 10000
PAD_NODES = 10240
D = 128
NC = 2
NS = 16
NW = NC * NS
CHUNK = 80
LANES = 16
NMETA = 4


def _scale_rows(rows_ref, wb, m, i):
    del i

    def grp_body(g, carry):
        w16 = wb[m, 0, pl.ds(g * LANES, LANES)]
        for j in range(LANES):
            wj = jnp.broadcast_to(w16[j], (LANES,))
            row = g * LANES + j
            for v in range(D // LANES):
                sl = pl.ds(v * LANES, LANES)
                rows_ref[row, sl] = rows_ref[row, sl] * wj
        return carry

    lax.fori_loop(0, CHUNK // LANES, grp_body, 0)


def _sc_spmm_body(x_hbm, ei_hbm, w_hbm, out_hbm,
                  srcb, dstb, wb, rows, acc_sh,
                  sem_m, sem_g, sem_s):
    c = lax.axis_index("c")
    s = lax.axis_index("s")
    wid = c * NS + s
    n_chunks = ei_hbm.shape[2]
    rows_per_tile = PAD_NODES // NS

    zv = jnp.zeros((LANES,), jnp.float32)

    def zero_body(j, carry):
        for v in range(D // LANES):
            rows[0, j, pl.ds(v * LANES, LANES)] = zv
        return carry

    lax.fori_loop(0, CHUNK, zero_body, 0)
    for q in range(rows_per_tile // CHUNK):
        pltpu.sync_copy(rows.at[0],
                        acc_sh.at[pl.ds(s * rows_per_tile + q * CHUNK, CHUNK)])
    rem = rows_per_tile % CHUNK
    if rem:
        base = rows_per_tile - rem
        pltpu.sync_copy(rows.at[0, pl.ds(0, rem)],
                        acc_sh.at[pl.ds(s * rows_per_tile + base, rem)])
    plsc.subcore_barrier()

    def start_meta(i, m):
        pltpu.async_copy(ei_hbm.at[0, wid, i], srcb.at[m], sem_m.at[m])
        pltpu.async_copy(ei_hbm.at[1, wid, i], dstb.at[m], sem_m.at[m])
        pltpu.async_copy(w_hbm.at[wid, i], wb.at[m], sem_m.at[m])

    def wait_meta(i, m):
        pltpu.make_async_copy(ei_hbm.at[0, wid, i], srcb.at[m],
                              sem_m.at[m]).wait()
        pltpu.make_async_copy(ei_hbm.at[1, wid, i], dstb.at[m],
                              sem_m.at[m]).wait()
        pltpu.make_async_copy(w_hbm.at[wid, i], wb.at[m], sem_m.at[m]).wait()

    half = CHUNK // 2

    def start_gather(m, p):
        for h in range(2):
            pltpu.async_copy(x_hbm.at[srcb.at[m, h]],
                             rows.at[p, pl.ds(h * half, half)], sem_g.at[p])

    def wait_gather(m, p):
        for h in range(2):
            pltpu.make_async_copy(x_hbm.at[srcb.at[m, h]],
                                  rows.at[p, pl.ds(h * half, half)],
                                  sem_g.at[p]).wait()

    def start_scatter(m, p):
        for h in range(2):
            pltpu.async_copy(rows.at[p, pl.ds(h * half, half)],
                             acc_sh.at[dstb.at[m, h]], sem_s.at[p],
                             add=True)

    def wait_scatter(m, p):
        for h in range(2):
            pltpu.make_async_copy(rows.at[p, pl.ds(h * half, half)],
                                  acc_sh.at[dstb.at[m, h]],
                                  sem_s.at[p]).wait()

    for j in range(NMETA - 1):
        start_meta(j, j)
    wait_meta(0, 0)
    start_gather(0, 0)
    wait_meta(1, 1)
    start_gather(1, 1)
    start_meta(NMETA - 1, NMETA - 1)
    wait_gather(0, 0)
    _scale_rows(rows.at[0], wb, 0, 0)
    start_scatter(0, 0)

    def quad_body(k, carry):
        for o in range(1, 5):
            i = 4 * k + o
            m = o % NMETA
            p = o % 2
            wait_scatter((o - 1) % NMETA, 1 - p)

            @pl.when(i + 3 < n_chunks)
            def _():
                start_meta(i + 3, (o + 3) % NMETA)

            @pl.when(i + 1 < n_chunks)
            def _():
                wait_meta(i + 1, (o + 1) % NMETA)
                start_gather((o + 1) % NMETA, 1 - p)

            wait_gather(m, p)
            _scale_rows(rows.at[p], wb, m, i)
            start_scatter(m, p)
        return carry

    lax.fori_loop(0, (n_chunks - 1) // 4, quad_body, 0)
    wait_scatter((n_chunks - 1) % NMETA, (n_chunks - 1) % 2)

    plsc.subcore_barrier()
    pltpu.sync_copy(acc_sh.at[pl.ds(s * rows_per_tile, rows_per_tile)],
                    out_hbm.at[c, pl.ds(s * rows_per_tile, rows_per_tile)])


def _sc_spmm(x, ei5, w4):
    mesh = plsc.VectorSubcoreMesh(core_axis_name="c", subcore_axis_name="s")
    return pl.kernel(
        _sc_spmm_body,
        out_type=jax.ShapeDtypeStruct((NC, PAD_NODES, D), jnp.float32),
        mesh=mesh,
        scratch_types=[
            pltpu.VMEM((NMETA, 2, CHUNK // 2), jnp.int32),
            pltpu.VMEM((NMETA, 2, CHUNK // 2), jnp.int32),
            pltpu.VMEM((NMETA, 1, CHUNK), jnp.float32),
            pltpu.VMEM((2, CHUNK, D), jnp.float32),
            pltpu.VMEM_SHARED((PAD_NODES, D), jnp.float32),
            pltpu.SemaphoreType.DMA((NMETA,)),
            pltpu.SemaphoreType.DMA((2,)),
            pltpu.SemaphoreType.DMA((2,)),
        ],
    )(x, ei5, w4)


def _tc_linear_body(p_ref, w_ref, b_ref, o_ref):
    a = p_ref[0] + p_ref[1]
    o_ref[...] = lax.dot_general(
        a, w_ref[...], (((1,), (1,)), ((), ())),
        preferred_element_type=jnp.float32) + b_ref[...]


def _tc_linear(parts, W, b2d):
    blk = 2000
    grid = N_NODES // blk
    return pl.pallas_call(
        _tc_linear_body,
        grid=(grid,),
        in_specs=[
            pl.BlockSpec((NC, blk, D), lambda i: (0, i, 0)),
            pl.BlockSpec((D, D), lambda i: (0, 0)),
            pl.BlockSpec((1, D), lambda i: (0, 0)),
        ],
        out_specs=pl.BlockSpec((blk, D), lambda i: (i, 0)),
        out_shape=jax.ShapeDtypeStruct((N_NODES, D), jnp.float32),
    )(parts, W, b2d)


def kernel(x, edge_index, edge_weight, W_weight, W_bias):
    n_edges = edge_index.shape[1]
    n_chunks = n_edges // (NW * CHUNK)
    ei5 = edge_index.astype(jnp.int32).reshape(2, NW, n_chunks, 2, CHUNK // 2)
    w4 = edge_weight.reshape(NW, n_chunks, 1, CHUNK)
    parts = _sc_spmm(x, ei5, w4)
    return _tc_linear(parts, W_weight, W_bias.reshape(1, D))


# speedup vs baseline: 1.0270x; 1.0270x over previous
import jax
import jax.numpy as jnp
from jax import lax
from jax.experimental import pallas as pl
from jax.experimental.pallas import tpu as pltpu
from jax.experimental.pallas import tpu_sc as plsc

N_NODES = 10000
PAD_NODES = 10240
D = 128
NC = 2
NS = 16
NW = NC * NS
CHUNK = 176
LANES = 16
NMETA = 4


def _scale_rows(rows_ref, wb, m, i):
    del i

    def grp_body(g, carry):
        w16 = wb[m, 0, pl.ds(g * LANES, LANES)]
        for j in range(LANES):
            wj = jnp.broadcast_to(w16[j], (LANES,))
            row = g * LANES + j
            for v in range(D // LANES):
                sl = pl.ds(v * LANES, LANES)
                rows_ref[row, sl] = rows_ref[row, sl] * wj
        return carry

    lax.fori_loop(0, CHUNK // LANES, grp_body, 0)


def _sc_spmm_body(x_hbm, ei_hbm, w_hbm, out_hbm,
                  srcb, dstb, wb, rows, acc_sh,
                  sem_m, sem_g, sem_s):
    c = lax.axis_index("c")
    s = lax.axis_index("s")
    wid = c * NS + s
    n_chunks = ei_hbm.shape[2]
    rows_per_tile = PAD_NODES // NS

    zv = jnp.zeros((LANES,), jnp.float32)

    def zero_body(j, carry):
        for v in range(D // LANES):
            rows[0, j, pl.ds(v * LANES, LANES)] = zv
        return carry

    lax.fori_loop(0, CHUNK, zero_body, 0)
    for q in range(rows_per_tile // CHUNK):
        pltpu.sync_copy(rows.at[0],
                        acc_sh.at[pl.ds(s * rows_per_tile + q * CHUNK, CHUNK)])
    rem = rows_per_tile % CHUNK
    if rem:
        base = rows_per_tile - rem
        pltpu.sync_copy(rows.at[0, pl.ds(0, rem)],
                        acc_sh.at[pl.ds(s * rows_per_tile + base, rem)])
    plsc.subcore_barrier()

    def start_meta(i, m):
        pltpu.async_copy(ei_hbm.at[0, wid, i], srcb.at[m], sem_m.at[m])
        pltpu.async_copy(ei_hbm.at[1, wid, i], dstb.at[m], sem_m.at[m])
        pltpu.async_copy(w_hbm.at[wid, i], wb.at[m], sem_m.at[m])

    def wait_meta(i, m):
        pltpu.make_async_copy(ei_hbm.at[0, wid, i], srcb.at[m],
                              sem_m.at[m]).wait()
        pltpu.make_async_copy(ei_hbm.at[1, wid, i], dstb.at[m],
                              sem_m.at[m]).wait()
        pltpu.make_async_copy(w_hbm.at[wid, i], wb.at[m], sem_m.at[m]).wait()

    def start_gather(m, p):
        pltpu.async_copy(x_hbm.at[srcb.at[m, 0]], rows.at[p], sem_g.at[p])

    def wait_gather(m, p):
        pltpu.make_async_copy(x_hbm.at[srcb.at[m, 0]], rows.at[p],
                              sem_g.at[p]).wait()

    def start_scatter(m, p):
        pltpu.async_copy(rows.at[p], acc_sh.at[dstb.at[m, 0]], sem_s.at[p],
                         add=True)

    def wait_scatter(m, p):
        pltpu.make_async_copy(rows.at[p], acc_sh.at[dstb.at[m, 0]],
                              sem_s.at[p]).wait()

    for j in range(NMETA - 1):
        start_meta(j, j)
    wait_meta(0, 0)
    start_gather(0, 0)
    wait_meta(1, 1)
    start_gather(1, 1)
    start_meta(NMETA - 1, NMETA - 1)
    wait_gather(0, 0)
    _scale_rows(rows.at[0], wb, 0, 0)
    start_scatter(0, 0)

    def quad_body(k, carry):
        for o in range(1, 5):
            i = 4 * k + o
            m = o % NMETA
            p = o % 2
            wait_scatter((o - 1) % NMETA, 1 - p)

            @pl.when(i + 3 < n_chunks)
            def _():
                start_meta(i + 3, (o + 3) % NMETA)

            @pl.when(i + 1 < n_chunks)
            def _():
                wait_meta(i + 1, (o + 1) % NMETA)
                start_gather((o + 1) % NMETA, 1 - p)

            wait_gather(m, p)
            _scale_rows(rows.at[p], wb, m, i)
            start_scatter(m, p)
        return carry

    lax.fori_loop(0, (n_chunks - 1) // 4, quad_body, 0)
    wait_scatter((n_chunks - 1) % NMETA, (n_chunks - 1) % 2)

    plsc.subcore_barrier()
    pltpu.sync_copy(acc_sh.at[pl.ds(s * rows_per_tile, rows_per_tile)],
                    out_hbm.at[c, pl.ds(s * rows_per_tile, rows_per_tile)])


def _sc_spmm(x, ei5, w4):
    mesh = plsc.VectorSubcoreMesh(core_axis_name="c", subcore_axis_name="s")
    return pl.kernel(
        _sc_spmm_body,
        out_type=jax.ShapeDtypeStruct((NC, PAD_NODES, D), jnp.float32),
        mesh=mesh,
        scratch_types=[
            pltpu.VMEM((NMETA, 1, CHUNK), jnp.int32),
            pltpu.VMEM((NMETA, 1, CHUNK), jnp.int32),
            pltpu.VMEM((NMETA, 1, CHUNK), jnp.float32),
            pltpu.VMEM((2, CHUNK, D), jnp.float32),
            pltpu.VMEM_SHARED((PAD_NODES, D), jnp.float32),
            pltpu.SemaphoreType.DMA((NMETA,)),
            pltpu.SemaphoreType.DMA((2,)),
            pltpu.SemaphoreType.DMA((2,)),
        ],
    )(x, ei5, w4)


def _tc_linear_body(p_ref, w_ref, b_ref, o_ref):
    a = p_ref[0] + p_ref[1]
    o_ref[...] = lax.dot_general(
        a, w_ref[...], (((1,), (1,)), ((), ())),
        preferred_element_type=jnp.float32) + b_ref[...]


def _tc_linear(parts, W, b2d):
    blk = 2000
    grid = N_NODES // blk
    return pl.pallas_call(
        _tc_linear_body,
        grid=(grid,),
        in_specs=[
            pl.BlockSpec((NC, blk, D), lambda i: (0, i, 0)),
            pl.BlockSpec((D, D), lambda i: (0, 0)),
            pl.BlockSpec((1, D), lambda i: (0, 0)),
        ],
        out_specs=pl.BlockSpec((blk, D), lambda i: (i, 0)),
        out_shape=jax.ShapeDtypeStruct((N_NODES, D), jnp.float32),
    )(parts, W, b2d)


def kernel(x, edge_index, edge_weight, W_weight, W_bias):
    n_edges = edge_index.shape[1]
    per_tile = n_edges // NW
    n_chunks = -(-per_tile // CHUNK)
    pad = n_chunks * CHUNK - per_tile
    wid = jnp.arange(NW, dtype=jnp.int32)[:, None]
    src2 = edge_index[0].astype(jnp.int32).reshape(NW, per_tile)
    dst2 = edge_index[1].astype(jnp.int32).reshape(NW, per_tile)
    pad_src = jnp.broadcast_to(wid, (NW, pad))
    pad_dst = jnp.broadcast_to(N_NODES + wid * 7, (NW, pad))
    src2 = jnp.concatenate([src2, pad_src], axis=1)
    dst2 = jnp.concatenate([dst2, pad_dst], axis=1)
    ei5 = jnp.stack([src2, dst2]).reshape(2, NW, n_chunks, 1, CHUNK)
    w2 = jnp.concatenate(
        [edge_weight.reshape(NW, per_tile),
         jnp.zeros((NW, pad), jnp.float32)], axis=1)
    w4 = w2.reshape(NW, n_chunks, 1, CHUNK)
    parts = _sc_spmm(x, ei5, w4)
    return _tc_linear(parts, W_weight, W_bias.reshape(1, D))


# speedup vs baseline: 1.0415x; 1.0142x over previous
import jax
import jax.numpy as jnp
from jax import lax
from jax.experimental import pallas as pl
from jax.experimental.pallas import tpu as pltpu
from jax.experimental.pallas import tpu_sc as plsc

N_NODES = 10000
PAD_NODES = 10240
D = 128
NC = 2
NS = 16
NW = NC * NS
CHUNK = 80
LANES = 16
NMETA = 4


def _scale_rows(rows_ref, wb, m, i):
    del i

    def grp_body(g, carry):
        w16 = wb[m, 0, pl.ds(g * LANES, LANES)]
        for j in range(LANES):
            wj = jnp.broadcast_to(w16[j], (LANES,))
            row = g * LANES + j
            for v in range(D // LANES):
                sl = pl.ds(v * LANES, LANES)
                rows_ref[row, sl] = rows_ref[row, sl] * wj
        return carry

    lax.fori_loop(0, CHUNK // LANES, grp_body, 0)


def _sc_spmm_body(x_hbm, ei_hbm, w_hbm, out_hbm,
                  srcb, dstb, wb, rows, acc_sh,
                  sem_m, sem_g, sem_s):
    c = lax.axis_index("c")
    s = lax.axis_index("s")
    wid = c * NS + s
    n_chunks = ei_hbm.shape[2]
    rows_per_tile = PAD_NODES // NS

    zv = jnp.zeros((LANES,), jnp.float32)

    def zero_body(j, carry):
        for v in range(D // LANES):
            rows[0, j, pl.ds(v * LANES, LANES)] = zv
        return carry

    lax.fori_loop(0, CHUNK, zero_body, 0)
    for q in range(rows_per_tile // CHUNK):
        pltpu.sync_copy(rows.at[0],
                        acc_sh.at[pl.ds(s * rows_per_tile + q * CHUNK, CHUNK)])
    plsc.subcore_barrier()

    def start_meta(i, m):
        pltpu.async_copy(ei_hbm.at[0, wid, i], srcb.at[m], sem_m.at[m])
        pltpu.async_copy(ei_hbm.at[1, wid, i], dstb.at[m], sem_m.at[m])
        pltpu.async_copy(w_hbm.at[wid, i], wb.at[m], sem_m.at[m])

    def wait_meta(i, m):
        pltpu.make_async_copy(ei_hbm.at[0, wid, i], srcb.at[m],
                              sem_m.at[m]).wait()
        pltpu.make_async_copy(ei_hbm.at[1, wid, i], dstb.at[m],
                              sem_m.at[m]).wait()
        pltpu.make_async_copy(w_hbm.at[wid, i], wb.at[m], sem_m.at[m]).wait()

    def start_gather(m, p):
        pltpu.async_copy(x_hbm.at[srcb.at[m, 0]], rows.at[p], sem_g.at[p])

    def wait_gather(m, p):
        pltpu.make_async_copy(x_hbm.at[srcb.at[m, 0]], rows.at[p],
                              sem_g.at[p]).wait()

    def start_scatter(m, p):
        pltpu.async_copy(rows.at[p], acc_sh.at[dstb.at[m, 0]], sem_s.at[p],
                         add=True)

    def wait_scatter(m, p):
        pltpu.make_async_copy(rows.at[p], acc_sh.at[dstb.at[m, 0]],
                              sem_s.at[p]).wait()

    for j in range(NMETA - 1):
        start_meta(j, j)
    wait_meta(0, 0)
    start_gather(0, 0)
    wait_meta(1, 1)
    start_gather(1, 1)
    start_meta(NMETA - 1, NMETA - 1)
    wait_gather(0, 0)
    _scale_rows(rows.at[0], wb, 0, 0)
    start_scatter(0, 0)

    def quad_body(k, carry):
        for o in range(1, 5):
            i = 4 * k + o
            m = o % NMETA
            p = o % 2
            wait_scatter((o - 1) % NMETA, 1 - p)

            @pl.when(i + 3 < n_chunks)
            def _():
                start_meta(i + 3, (o + 3) % NMETA)

            @pl.when(i + 1 < n_chunks)
            def _():
                wait_meta(i + 1, (o + 1) % NMETA)
                start_gather((o + 1) % NMETA, 1 - p)

            wait_gather(m, p)
            _scale_rows(rows.at[p], wb, m, i)
            start_scatter(m, p)
        return carry

    lax.fori_loop(0, (n_chunks - 1) // 4, quad_body, 0)
    wait_scatter((n_chunks - 1) % NMETA, (n_chunks - 1) % 2)

    plsc.subcore_barrier()
    pltpu.sync_copy(acc_sh.at[pl.ds(s * rows_per_tile, rows_per_tile)],
                    out_hbm.at[c, pl.ds(s * rows_per_tile, rows_per_tile)])


def _sc_spmm(x, ei5, w4):
    mesh = plsc.VectorSubcoreMesh(core_axis_name="c", subcore_axis_name="s")
    return pl.kernel(
        _sc_spmm_body,
        out_type=jax.ShapeDtypeStruct((NC, PAD_NODES, D), jnp.float32),
        mesh=mesh,
        scratch_types=[
            pltpu.VMEM((NMETA, 1, CHUNK), jnp.int32),
            pltpu.VMEM((NMETA, 1, CHUNK), jnp.int32),
            pltpu.VMEM((NMETA, 1, CHUNK), jnp.float32),
            pltpu.VMEM((2, CHUNK, D), jnp.float32),
            pltpu.VMEM_SHARED((PAD_NODES, D), jnp.float32),
            pltpu.SemaphoreType.DMA((NMETA,)),
            pltpu.SemaphoreType.DMA((2,)),
            pltpu.SemaphoreType.DMA((2,)),
        ],
    )(x, ei5, w4)


def _tc_linear_body(p_ref, w_ref, b_ref, o_ref):
    a = p_ref[0] + p_ref[1]
    o_ref[...] = lax.dot_general(
        a, w_ref[...], (((1,), (1,)), ((), ())),
        preferred_element_type=jnp.float32) + b_ref[...]


def _tc_linear(parts, W, b2d):
    blk = 2000
    grid = N_NODES // blk
    return pl.pallas_call(
        _tc_linear_body,
        grid=(grid,),
        in_specs=[
            pl.BlockSpec((NC, blk, D), lambda i: (0, i, 0)),
            pl.BlockSpec((D, D), lambda i: (0, 0)),
            pl.BlockSpec((1, D), lambda i: (0, 0)),
        ],
        out_specs=pl.BlockSpec((blk, D), lambda i: (i, 0)),
        out_shape=jax.ShapeDtypeStruct((N_NODES, D), jnp.float32),
    )(parts, W, b2d)


def kernel(x, edge_index, edge_weight, W_weight, W_bias):
    n_edges = edge_index.shape[1]
    n_chunks = n_edges // (NW * CHUNK)
    ei5 = edge_index.astype(jnp.int32).reshape(2, NW, n_chunks, 1, CHUNK)
    w4 = edge_weight.reshape(NW, n_chunks, 1, CHUNK)
    parts = _sc_spmm(x, ei5, w4)
    return _tc_linear(parts, W_weight, W_bias.reshape(1, D))
